# baseline (speedup 1.0000x reference)
import jax
import jax.numpy as jnp
from jax import lax
from jax.experimental import pallas as pl
from jax.experimental.pallas import tpu as pltpu
from jax.experimental.pallas import tpu_sc as plsc

N_NODES = 10000
N_PAD = 10240
DIM = 256
E = 320000
NC, NS = 2, 16
NW = NC * NS
RPT = N_PAD // NW
RPSC = N_PAD // NC
MAGIC = 6554
CAP = 16512
CB = 2048
E_PAD = 321536
NCH_BIN = E_PAD // CB
CHUNK = 64
FB = 40
PAD_NODE = N_PAD - 1


def _worker_id():
    return lax.axis_index("c") * NS + lax.axis_index("s")


def _bin_body(col_hbm, row_hbm, rows_out, lcol_out, cnt_out, deg_out,
              colv, rowv, rbuf, lbuf, degv, cntv, sem_c, sem_r):
    o = _worker_id()
    base = o * RPT
    base_sc = lax.axis_index("c") * RPSC

    zf = jnp.zeros((16,), jnp.float32)
    for i in range(RPT // 16):
        degv[pl.ds(i * 16, 16)] = zf

    def issue(k):
        b = k & 1
        pltpu.async_copy(col_hbm.at[pl.ds(k * CB, CB)], colv.at[b], sem_c)
        pltpu.async_copy(row_hbm.at[pl.ds(k * CB, CB)], rowv.at[b], sem_r)

    issue(0)
    ones_f = jnp.ones((16,), jnp.float32)
    o_v = jnp.full((16,), o, jnp.int32)
    basesc_v = jnp.full((16,), base_sc, jnp.int32)
    base_v = jnp.full((16,), base, jnp.int32)
    magic_v = jnp.full((16,), MAGIC, jnp.int32)
    sh21_v = jnp.full((16,), 21, jnp.int32)
    one_v = jnp.ones((16,), jnp.int32)
    cap_v = jnp.full((16,), CAP - 1, jnp.int32)
    iota_v = lax.iota(jnp.int32, 16)

    def outer(k, cur):
        b = k & 1
        pltpu.make_async_copy(col_hbm.at[pl.ds(k * CB, CB)], colv.at[b],
                              sem_c).wait()
        pltpu.make_async_copy(row_hbm.at[pl.ds(k * CB, CB)], rowv.at[b],
                              sem_r).wait()

        @pl.when(k + 1 < NCH_BIN)
        def _():
            issue(k + 1)

        def inner(j, cur):
            cv = colv[b, pl.ds(j * 16, 16)]
            rv = rowv[b, pl.ds(j * 16, 16)]
            own = lax.shift_right_logical(cv * magic_v, sh21_v)
            m = own == o_v
            lc = cv - base_v
            lsc = cv - basesc_v
            pos = cur + plsc.cumsum(m.astype(jnp.int32)) - one_v
            pos = jnp.minimum(pos, cap_v)
            plsc.store_scatter(rbuf, [pos], rv, mask=m)
            plsc.store_scatter(lbuf, [pos], lc, mask=m)
            plsc.addupdate_scatter(degv, [lc], ones_f, mask=m)
            return cur + plsc.all_reduce_population_count(m)

        return lax.fori_loop(0, CB // 16, inner, cur)

    cur = lax.fori_loop(0, NCH_BIN, outer, jnp.zeros((16,), jnp.int32))

    zi = jnp.full((16,), PAD_NODE, jnp.int32)
    dmp = jnp.zeros((16,), jnp.int32)
    for t in range(CHUNK // 16):
        off_v = jnp.full((16,), t * 16, jnp.int32)
        pos = jnp.minimum(cur + off_v + iota_v, cap_v)
        plsc.store_scatter(rbuf, [pos], zi)
        plsc.store_scatter(lbuf, [pos], dmp)
    cur = jnp.bitwise_and(cur + jnp.full((16,), CHUNK, jnp.int32),
                          jnp.full((16,), -CHUNK, jnp.int32))

    cntv[...] = cur
    pltpu.sync_copy(cntv, cnt_out.at[o])
    pltpu.sync_copy(rbuf, rows_out.at[o])
    pltpu.sync_copy(lbuf, lcol_out.at[o])
    pltpu.sync_copy(degv, deg_out.at[o])


def _bin(col, row):
    mesh = plsc.VectorSubcoreMesh(core_axis_name="c", subcore_axis_name="s")
    f = pl.kernel(
        _bin_body,
        out_type=(
            jax.ShapeDtypeStruct((NW, CAP), jnp.int32),
            jax.ShapeDtypeStruct((NW, CAP), jnp.int32),
            jax.ShapeDtypeStruct((NW, 16), jnp.int32),
            jax.ShapeDtypeStruct((NW, RPT), jnp.float32),
        ),
        mesh=mesh,
        scratch_types=[
            pltpu.VMEM((2, CB), jnp.int32),
            pltpu.VMEM((2, CB), jnp.int32),
            pltpu.VMEM((CAP,), jnp.int32),
            pltpu.VMEM((CAP,), jnp.int32),
            pltpu.VMEM((RPT,), jnp.float32),
            pltpu.VMEM((16,), jnp.int32),
            pltpu.SemaphoreType.DMA,
            pltpu.SemaphoreType.DMA,
        ],
        compiler_params=pltpu.CompilerParams(needs_layout_passes=False),
    )
    return f(col, row)


def _layer_body(z_hbm, rowl, lcoll, cnt_hbm, dis2_hbm, zeros_hbm, zout,
                acc, gbuf, ridx0, ridx1, cidx0, cidx1,
                cntv, dis2v, sem_ir, sem_ic, sem_g):
    o = _worker_id()
    base = o * RPT
    rslots = (ridx0, ridx1)
    cslots = (cidx0, cidx1)

    pltpu.sync_copy(cnt_hbm.at[o], cntv)
    pltpu.sync_copy(zeros_hbm, acc)
    pltpu.sync_copy(dis2_hbm.at[pl.ds(base, RPT)], dis2v)
    nch = jnp.max(cntv[...]) // CHUNK

    def rsel(g, fn):
        for i, ref in enumerate(rslots):
            @pl.when(lax.rem(g, 2) == i)
            def _(ref=ref):
                fn(ref, i)

    def a_start(g):
        rsel(g, lambda r, i: pltpu.async_copy(
            rowl.at[o, pl.ds(g * CHUNK, CHUNK)], r, sem_ir))
        rsel(g, lambda _, i: pltpu.async_copy(
            lcoll.at[o, pl.ds(g * CHUNK, CHUNK)], cslots[i], sem_ic))

    def a_wait(g):
        rsel(g, lambda r, i: pltpu.make_async_copy(
            rowl.at[o, pl.ds(g * CHUNK, CHUNK)], r, sem_ir).wait())
        rsel(g, lambda _, i: pltpu.make_async_copy(
            lcoll.at[o, pl.ds(g * CHUNK, CHUNK)], cslots[i], sem_ic).wait())

    def b_start(g):
        def start4(r, i):
            for q in range(4):
                pltpu.async_copy(z_hbm.at[r.at[pl.ds(q * 16, 16)]],
                                 gbuf.at[i, pl.ds(q * 16, 16)], sem_g)
        rsel(g, start4)

    def b_wait(g):
        def wait4(r, i):
            for q in range(4):
                pltpu.make_async_copy(z_hbm.at[r.at[pl.ds(q * 16, 16)]],
                                      gbuf.at[i, pl.ds(q * 16, 16)],
                                      sem_g).wait()
        rsel(g, wait4)

    iota_v = lax.iota(jnp.int32, 16)
    one_v = jnp.ones((16,), jnp.int32)

    def accum(g):
        def do(_, i):
            slot_v = jnp.full((16,), i, jnp.int32)
            for grp in range(CHUNK // 16):
                lcol = cslots[i][pl.ds(grp * 16, 16)]
                srow_v = jnp.full((16,), grp * 16, jnp.int32) + iota_v

                def kbody(k, carry):
                    kv = jnp.full((16,), k, jnp.int32)
                    g16 = plsc.load_gather(gbuf, [slot_v, srow_v, kv])
                    plsc.addupdate_scatter(acc, [lcol, kv], g16)
                    return carry

                lax.fori_loop(0, DIM, kbody, 0, unroll=16)

        rsel(g, do)

    a_start(0)

    @pl.when(nch > 1)
    def _():
        a_start(1)

    a_wait(0)
    b_start(0)

    def step(g, carry):
        b_wait(g)

        @pl.when(g + 1 < nch)
        def _():
            a_wait(g + 1)
            b_start(g + 1)

        accum(g)

        @pl.when(g + 2 < nch)
        def _():
            a_start(g + 2)

        return carry

    lax.fori_loop(0, nch, step, 0)

    def scale_row(r, carry):
        d2 = plsc.load_gather(dis2v, [jnp.full((16,), r, jnp.int32)])
        for jj in range(DIM // 16):
            sl = pl.ds(jj * 16, 16)
            acc[r, sl] = acc[r, sl] * d2
        return carry

    lax.fori_loop(0, RPT, scale_row, 0)
    pltpu.sync_copy(acc, zout.at[pl.ds(base, RPT)])


def _layer(z, rowl, lcoll, cnt, dis2, zeros_buf):
    mesh = plsc.VectorSubcoreMesh(core_axis_name="c", subcore_axis_name="s")
    f = pl.kernel(
        _layer_body,
        out_type=jax.ShapeDtypeStruct((N_PAD, DIM), jnp.float32),
        mesh=mesh,
        scratch_types=[
            pltpu.VMEM((RPT, DIM), jnp.float32),
            pltpu.VMEM((2, CHUNK, DIM), jnp.float32),
            pltpu.VMEM((CHUNK,), jnp.int32),
            pltpu.VMEM((CHUNK,), jnp.int32),
            pltpu.VMEM((CHUNK,), jnp.int32),
            pltpu.VMEM((CHUNK,), jnp.int32),
            pltpu.VMEM((16,), jnp.int32),
            pltpu.VMEM((RPT,), jnp.float32),
            pltpu.SemaphoreType.DMA,
            pltpu.SemaphoreType.DMA,
            pltpu.SemaphoreType.DMA,
        ],
        compiler_params=pltpu.CompilerParams(needs_layout_passes=False),
    )
    return f(z, rowl, lcoll, cnt, dis2, zeros_buf)


def _scale_body(x0_ref, dis_ref, z1_ref):
    i = pl.program_id(0)
    d = dis_ref[pl.ds(i * 512, 512), :]
    z1_ref[...] = x0_ref[...] * d


def _epi_body(x0_ref, z2_ref, z3_ref, z4_ref, invd_ref, out_ref):
    i = pl.program_id(0)
    inv = invd_ref[pl.ds(i * 512, 512), :]
    s = z2_ref[...] + z3_ref[...] + z4_ref[...]
    out_ref[...] = 0.25 * (x0_ref[...] + s * inv)


_BLK = pl.BlockSpec((512, DIM), lambda i: (i, 0))
_FULL1 = pl.BlockSpec((N_PAD, 1), lambda i: (0, 0))


def _scale(x0, dis):
    return pl.pallas_call(
        _scale_body,
        grid=(N_PAD // 512,),
        in_specs=[_BLK, _FULL1],
        out_specs=_BLK,
        out_shape=jax.ShapeDtypeStruct((N_PAD, DIM), jnp.float32),
    )(x0, dis)


def _epilogue(x0, z2, z3, z4, invd):
    return pl.pallas_call(
        _epi_body,
        grid=(N_PAD // 512,),
        in_specs=[_BLK, _BLK, _BLK, _BLK, _FULL1],
        out_specs=_BLK,
        out_shape=jax.ShapeDtypeStruct((N_PAD, DIM), jnp.float32),
    )(x0, z2, z3, z4, invd)


def kernel(edge_index, user_emb, item_emb):
    n_users = user_emb.shape[0]
    row = edge_index[0].astype(jnp.int32)
    col = edge_index[1].astype(jnp.int32)
    row = jnp.pad(row, (0, E_PAD - E), constant_values=N_PAD - 1)
    col = jnp.pad(col, (0, E_PAD - E), constant_values=N_PAD - 1)

    rows_l, lcol_l, cnt, deg = _bin(col, row)
    deg_f = deg.reshape(N_PAD, 1)
    dis = jnp.where(deg_f > 0, lax.rsqrt(deg_f), 0.0)
    invd = jnp.where(deg_f > 0, 1.0 / dis, 0.0)
    dis2 = jnp.where(deg_f > 0, 1.0 / deg_f, 0.0).reshape(N_PAD)

    x0 = jnp.concatenate(
        [user_emb, item_emb,
         jnp.zeros((N_PAD - N_NODES, DIM), jnp.float32)], axis=0)
    z1 = _scale(x0, dis)

    zeros_buf = jnp.zeros((RPT, DIM), jnp.float32)
    z2 = _layer(z1, rows_l, lcol_l, cnt, dis2, zeros_buf)
    z3 = _layer(z2, rows_l, lcol_l, cnt, dis2, zeros_buf)
    z4 = _layer(z3, rows_l, lcol_l, cnt, dis2, zeros_buf)

    final = _epilogue(x0, z2, z3, z4, invd)
    return (final[:n_users], final[n_users:N_NODES])


# speedup vs baseline: 1.4496x; 1.4496x over previous
import functools

import jax
import jax.numpy as jnp
from jax.experimental import pallas as pl

N_NODES = 10000
N_PAD = 10240
DIM = 256
BM = 256


def _mm_body(x0_ref, y_ref, a_ref, out_ref, *, scale):
    acc = jnp.dot(a_ref[...], y_ref[...].astype(jnp.bfloat16),
                  preferred_element_type=jnp.float32)
    out_ref[...] = (x0_ref[...] + acc) * scale


def _propagate(a, x0, y, scale):
    grid = (N_PAD // BM,)
    return pl.pallas_call(
        functools.partial(_mm_body, scale=scale),
        grid=grid,
        in_specs=[
            pl.BlockSpec((BM, DIM), lambda i: (i, 0)),
            pl.BlockSpec((N_PAD, DIM), lambda i: (0, 0)),
            pl.BlockSpec((BM, N_PAD), lambda i: (i, 0)),
        ],
        out_specs=pl.BlockSpec((BM, DIM), lambda i: (i, 0)),
        out_shape=jax.ShapeDtypeStruct((N_PAD, DIM), jnp.float32),
    )(x0, y, a)


def kernel(edge_index, user_emb, item_emb):
    n_users = user_emb.shape[0]
    row = edge_index[0].astype(jnp.int32)
    col = edge_index[1].astype(jnp.int32)

    deg = jnp.zeros((N_PAD,), jnp.float32).at[col].add(1.0)
    dis = jnp.where(deg > 0, jax.lax.rsqrt(deg), 0.0)
    norm = dis[row] * dis[col]

    a = jnp.zeros((N_PAD, N_PAD), jnp.bfloat16).at[col, row].add(
        norm.astype(jnp.bfloat16))

    x0 = jnp.concatenate(
        [user_emb, item_emb,
         jnp.zeros((N_PAD - N_NODES, DIM), jnp.float32)], axis=0)

    y = _propagate(a, x0, x0, 1.0)
    y = _propagate(a, x0, y, 1.0)
    final = _propagate(a, x0, y, 0.25)

    return (final[:n_users], final[n_users:N_NODES])


# speedup vs baseline: 1.8519x; 1.2776x over previous
import functools

import jax
import jax.numpy as jnp
from jax.experimental import pallas as pl

N_NODES = 10000
N_PAD = 10240
DIM = 256
BM = 256


def _mm_body(x0_ref, y_ref, a_ref, out_ref, *, scale):
    acc = jnp.dot(a_ref[...], y_ref[...], preferred_element_type=jnp.float32)
    out_ref[...] = (x0_ref[...] + acc) * scale


def _propagate(a, x0, y, scale):
    grid = (N_PAD // BM,)
    return pl.pallas_call(
        functools.partial(_mm_body, scale=scale),
        grid=grid,
        in_specs=[
            pl.BlockSpec((BM, DIM), lambda i: (i, 0)),
            pl.BlockSpec((N_PAD, DIM), lambda i: (0, 0)),
            pl.BlockSpec((BM, N_PAD), lambda i: (i, 0)),
        ],
        out_specs=pl.BlockSpec((BM, DIM), lambda i: (i, 0)),
        out_shape=jax.ShapeDtypeStruct((N_PAD, DIM), jnp.float32),
    )(x0, y, a)


def kernel(edge_index, user_emb, item_emb):
    n_users = user_emb.shape[0]
    row = edge_index[0].astype(jnp.int32)
    col = edge_index[1].astype(jnp.int32)

    deg = jnp.zeros((N_PAD,), jnp.float32).at[col].add(1.0)
    dis = jnp.where(deg > 0, jax.lax.rsqrt(deg), 0.0)
    norm = dis[row] * dis[col]

    a = jnp.zeros((N_PAD, N_PAD), jnp.float32).at[col, row].add(norm)

    x0 = jnp.concatenate(
        [user_emb, item_emb,
         jnp.zeros((N_PAD - N_NODES, DIM), jnp.float32)], axis=0)

    y = _propagate(a, x0, x0, 1.0)
    y = _propagate(a, x0, y, 1.0)
    final = _propagate(a, x0, y, 0.25)

    return (final[:n_users], final[n_users:N_NODES])
